# baseline (speedup 1.0000x reference)
import functools

import jax
import jax.numpy as jnp
from jax import lax
from jax.experimental import pallas as pl
from jax.experimental.pallas import tpu as pltpu
from jax.experimental.pallas import tpu_sc as plsc


_N = 51200000
_NW = 32
_SH = _N // _NW
_CH = 100000

_mesh = plsc.VectorSubcoreMesh(core_axis_name="c", subcore_axis_name="s")


@functools.partial(
    pl.kernel,
    mesh=_mesh,
    out_type=jax.ShapeDtypeStruct((_N,), jnp.float32),
    scratch_types=[pltpu.VMEM((_CH,), jnp.float32)],
)
def _sc_memset(out_hbm, buf):
    wid = lax.axis_index("s") * 2 + lax.axis_index("c")

    def zero_body(i, carry):
        buf[pl.ds(i * 16, 16)] = jnp.zeros((16,), jnp.float32)
        return carry

    lax.fori_loop(0, _CH // 16, zero_body, 0)

    base = wid * _SH

    def dma_body(j, carry):
        pltpu.sync_copy(buf, out_hbm.at[pl.ds(base + j * _CH, _CH)])
        return carry

    lax.fori_loop(0, _SH // _CH, dma_body, 0)


def kernel(inputs):
    del inputs
    return _sc_memset()


# speedup vs baseline: 1.6366x; 1.6366x over previous
import jax
import jax.numpy as jnp
from jax import lax
from jax.experimental import pallas as pl


_B, _S, _C = 1024, 50, 1000
_CB = 64


def _onehot_t_body(xt_ref, out_ref):
    c0 = pl.program_id(0) * _CB
    xt = xt_ref[...]
    cvals = c0 + lax.broadcasted_iota(jnp.int32, (_S, _CB, _B), 1)
    out_ref[...] = (xt[:, None, :] == cvals).astype(jnp.float32)


def kernel(inputs):
    xt = inputs.astype(jnp.int32).T
    out_t = pl.pallas_call(
        _onehot_t_body,
        grid=(-(-_C // _CB),),
        in_specs=[pl.BlockSpec((_S, _B), lambda i: (0, 0))],
        out_specs=pl.BlockSpec((_S, _CB, _B), lambda i: (0, i, 0)),
        out_shape=jax.ShapeDtypeStruct((_S, _C, _B), jnp.float32),
    )(xt)
    return out_t.transpose(2, 0, 1)
